# Initial kernel scaffold; baseline (speedup 1.0000x reference)
import jax
import jax.numpy as jnp
from jax.experimental import pallas as pl


def kernel(A_data, V_data, output_data, input_data, gnn_layers, layer_parameters, output_lengths, input_lengths, params):
    raise NotImplementedError("write your pallas kernel here")


# speedup vs baseline: 2.5170x; 2.5170x over previous
import functools

import jax
import jax.numpy as jnp
from jax.experimental import pallas as pl

B, L, P, T = 16, 32, 32, 512
H, R, OUT = 1024, 1024, 1024

_DN = (((1,), (1,)), ((), ()))


def _dott(x, w):
    return jax.lax.dot_general(x, w, _DN, preferred_element_type=jnp.float32)


def _layer_paths_kernel(A_ref, V_ref, gl_ref, lp_ref,
                        twtbA_ref, W1A_ref, W2A_ref, W3A_ref, b1A_ref, b2A_ref, b3A_ref,
                        twtbV_ref, W1V_ref, W2V_ref, W3V_ref, b1V_ref, b2V_ref, b3V_ref,
                        Wih_ref, bih_ref, out_ref):
    gl = gl_ref[...]
    lp = lp_ref[...]
    lmask = (jax.lax.broadcasted_iota(jnp.int32, (B, L), 1) < gl).astype(jnp.float32)
    c_bl = lmask * lp.astype(jnp.float32)
    pmask = (jax.lax.broadcasted_iota(jnp.int32, (B, L, P), 2)
             < lp[:, :, None]).astype(jnp.float32)

    def path(data_ref, twtb_ref, W1_ref, W2_ref, W3_ref, b1_ref, b2_ref, b3_ref, wih):
        d = data_ref[...]
        s_bl = jnp.sum(d * pmask, axis=2) * lmask
        uv = _dott(twtb_ref[...], W1_ref[...])
        h1 = (s_bl[:, :, None] * uv[0:1, :][None]
              + c_bl[:, :, None] * uv[1:2, :][None]
              + b1_ref[...][None])
        h1 = jnp.maximum(h1, 0.0).reshape(B * L, H)
        h2 = jnp.maximum(_dott(h1, W2_ref[...]) + b2_ref[...], 0.0)
        enc = _dott(h2, W3_ref[...]) + b3_ref[...]
        return _dott(enc, wih)

    Wih = Wih_ref[...]
    xa = path(A_ref, twtbA_ref, W1A_ref, W2A_ref, W3A_ref, b1A_ref, b2A_ref, b3A_ref,
              Wih[:, :H])
    xv = path(V_ref, twtbV_ref, W1V_ref, W2V_ref, W3V_ref, b1V_ref, b2V_ref, b3V_ref,
              Wih[:, H:])
    out_ref[...] = xa + xv + bih_ref[...]


def _emb_paths_kernel(odata_ref, idata_ref, olen_ref, ilen_ref,
                      twtbO_ref, W1O_ref, W2O_ref, W3O_ref, b1O_ref, b2O_ref, b3O_ref,
                      twtbI_ref, W1I_ref, W2I_ref, W3I_ref, b1I_ref, b2I_ref, b3I_ref,
                      WoO_ref, Wsi_ref, bsi_ref, outpart_ref, state_ref):
    def path(data_ref, len_ref, twtb_ref, W1_ref, W2_ref, W3_ref, b1_ref, b2_ref, b3_ref):
        ln = len_ref[...]
        mask = (jax.lax.broadcasted_iota(jnp.int32, (B, T), 1) < ln).astype(jnp.float32)
        s = jnp.sum(data_ref[...] * mask, axis=1, keepdims=True)
        c = ln.astype(jnp.float32)
        uv = _dott(twtb_ref[...], W1_ref[...])
        h1 = jnp.maximum(s * uv[0:1, :] + c * uv[1:2, :] + b1_ref[...], 0.0)
        h2 = jnp.maximum(_dott(h1, W2_ref[...]) + b2_ref[...], 0.0)
        return _dott(h2, W3_ref[...]) + b3_ref[...]

    out_emb = path(odata_ref, olen_ref, twtbO_ref, W1O_ref, W2O_ref, W3O_ref,
                   b1O_ref, b2O_ref, b3O_ref)
    in_emb = path(idata_ref, ilen_ref, twtbI_ref, W1I_ref, W2I_ref, W3I_ref,
                  b1I_ref, b2I_ref, b3I_ref)
    outpart_ref[...] = _dott(out_emb, WoO_ref[...])
    state_ref[...] = _dott(in_emb, Wsi_ref[...]) + bsi_ref[...]


def _rnn_final_kernel(xT_ref, state_ref, outpart_ref, gl_ref,
                      Whh_ref, bhh_ref, WoH_ref, bo_ref, out_ref):
    gl = gl_ref[...]
    Whh = Whh_ref[...]
    bhh = bhh_ref[...]

    def step(t, h):
        x_t = xT_ref[t]
        hn = jnp.tanh(x_t + _dott(h, Whh) + bhh)
        return jnp.where(gl > t, hn, h)

    h = jax.lax.fori_loop(0, L, step, state_ref[...])
    out_ref[...] = _dott(h, WoH_ref[...]) + outpart_ref[...] + bo_ref[...]


@functools.partial(jax.jit, static_argnames=())
def kernel(A_data, V_data, output_data, input_data, gnn_layers, layer_parameters,
           output_lengths, input_lengths, params):
    p = params
    f32 = jnp.float32
    gl = gnn_layers.astype(jnp.int32).reshape(B, 1)
    lp = layer_parameters.astype(jnp.int32).reshape(B, 1)
    olen = output_lengths.astype(jnp.int32).reshape(B, 1)
    ilen = input_lengths.astype(jnp.int32).reshape(B, 1)

    def twtb(n):
        return jnp.stack([p[n + '_tw'], p[n + '_tb']], axis=0)

    def row(v):
        return v.reshape(1, -1)

    xproj = pl.pallas_call(
        _layer_paths_kernel,
        out_shape=jax.ShapeDtypeStruct((B * L, R), f32),
    )(A_data, V_data, gl, lp,
      twtb('A'), p['A_W1'], p['A_W2'], p['A_W3'],
      row(p['A_b1']), row(p['A_b2']), row(p['A_b3']),
      twtb('V'), p['V_W1'], p['V_W2'], p['V_W3'],
      row(p['V_b1']), row(p['V_b2']), row(p['V_b3']),
      p['Wih'], row(p['bih']))

    outpart, state = pl.pallas_call(
        _emb_paths_kernel,
        out_shape=(jax.ShapeDtypeStruct((B, OUT), f32),
                   jax.ShapeDtypeStruct((B, R), f32)),
    )(output_data, input_data, olen, ilen,
      twtb('out'), p['out_W1'], p['out_W2'], p['out_W3'],
      row(p['out_b1']), row(p['out_b2']), row(p['out_b3']),
      twtb('inp'), p['inp_W1'], p['inp_W2'], p['inp_W3'],
      row(p['inp_b1']), row(p['inp_b2']), row(p['inp_b3']),
      p['Wo'][:, R:], p['Wsi'], row(p['bsi']))

    xT = xproj.reshape(B, L, R).transpose(1, 0, 2)

    final = pl.pallas_call(
        _rnn_final_kernel,
        out_shape=jax.ShapeDtypeStruct((B, OUT), f32),
    )(xT, state, outpart, gl,
      p['Whh'], row(p['bhh']), p['Wo'][:, :R], row(p['bo']))

    return final


# speedup vs baseline: 3.6027x; 1.4313x over previous
import functools

import jax
import jax.numpy as jnp
from jax.experimental import pallas as pl
from jax.experimental.pallas import tpu as pltpu

B, L, P, T = 16, 32, 32, 512
H, R, OUT = 1024, 1024, 1024
NBUF = 8

_DN = (((1,), (1,)), ((), ()))


def _dott(x, w):
    return jax.lax.dot_general(x, w, _DN, preferred_element_type=jnp.float32)


def _iota(shape, dim):
    return jax.lax.broadcasted_iota(jnp.int32, shape, dim)


def _mega_kernel(AT_ref, VT_ref, od_ref, id_ref,
                 glc_ref, glr_ref, lpr_ref, olen_ref, ilen_ref,
                 twtbA_ref, twtbV_ref, twtbO_ref, twtbI_ref,
                 b1A_ref, b2A_ref, b3A_ref, b1V_ref, b2V_ref, b3V_ref,
                 b1O_ref, b2O_ref, b3O_ref, b1I_ref, b2I_ref, b3I_ref,
                 bih_ref, bsi_ref, bhh_ref, bo_ref,
                 w1a, w1v, w1o, w1i, w2a, w3a, w2v, w3v,
                 w2o, w3o, w2i, w3i, wih, wo, wsi, whh,
                 out_ref, wbuf, xs, sems):
    srcs = [w1a, w1v, w1o, w1i,
            w2a, w3a, wih.at[:, :H],
            w2v, w3v, wih.at[:, H:],
            w2o, w3o, wo.at[:, R:],
            w2i, w3i, wsi,
            whh, wo.at[:, :R]]
    n_uses = len(srcs)

    def start(k):
        if k < n_uses:
            pltpu.make_async_copy(srcs[k], wbuf.at[k % NBUF], sems.at[k % NBUF]).start()

    def wait(k):
        pltpu.make_async_copy(srcs[k], wbuf.at[k % NBUF], sems.at[k % NBUF]).wait()
        return wbuf[k % NBUF]

    for k in range(NBUF):
        start(k)

    glr = glr_ref[...]
    lpr = lpr_ref[...]
    lmask = (_iota((L, B), 0) < glr).astype(jnp.float32)
    c_lb = lmask * lpr.astype(jnp.float32)
    pmask = (_iota((L, B, P), 2) < lpr[:, :, None]).astype(jnp.float32)
    sA = jnp.sum(AT_ref[...] * pmask, axis=2) * lmask
    sV = jnp.sum(VT_ref[...] * pmask, axis=2) * lmask

    olen = olen_ref[...]
    ilen = ilen_ref[...]
    omask = (_iota((B, T), 1) < olen).astype(jnp.float32)
    imask = (_iota((B, T), 1) < ilen).astype(jnp.float32)
    sO = jnp.sum(od_ref[...] * omask, axis=1, keepdims=True)
    sI = jnp.sum(id_ref[...] * imask, axis=1, keepdims=True)
    cO = olen.astype(jnp.float32)
    cI = ilen.astype(jnp.float32)

    w = wait(0); uvA = _dott(twtbA_ref[...], w); start(0 + NBUF)
    w = wait(1); uvV = _dott(twtbV_ref[...], w); start(1 + NBUF)
    w = wait(2); uvO = _dott(twtbO_ref[...], w); start(2 + NBUF)
    w = wait(3); uvI = _dott(twtbI_ref[...], w); start(3 + NBUF)

    def h1_layer(s_lb, uv, b1):
        h = (s_lb[:, :, None] * uv[0:1, :][None]
             + c_lb[:, :, None] * uv[1:2, :][None] + b1[...][None])
        return jnp.maximum(h, 0.0).reshape(L * B, H)

    h1 = h1_layer(sA, uvA, b1A_ref)
    w = wait(4); h2 = jnp.maximum(_dott(h1, w) + b2A_ref[...], 0.0); start(4 + NBUF)
    w = wait(5); enc = _dott(h2, w) + b3A_ref[...]; start(5 + NBUF)
    w = wait(6); xs[...] = _dott(enc, w) + bih_ref[...]; start(6 + NBUF)

    h1 = h1_layer(sV, uvV, b1V_ref)
    w = wait(7); h2 = jnp.maximum(_dott(h1, w) + b2V_ref[...], 0.0); start(7 + NBUF)
    w = wait(8); enc = _dott(h2, w) + b3V_ref[...]; start(8 + NBUF)
    w = wait(9); xs[...] = xs[...] + _dott(enc, w); start(9 + NBUF)

    h1e = jnp.maximum(sO * uvO[0:1, :] + cO * uvO[1:2, :] + b1O_ref[...], 0.0)
    w = wait(10); h2e = jnp.maximum(_dott(h1e, w) + b2O_ref[...], 0.0)
    w = wait(11); embO = _dott(h2e, w) + b3O_ref[...]
    w = wait(12); outpart = _dott(embO, w)

    h1e = jnp.maximum(sI * uvI[0:1, :] + cI * uvI[1:2, :] + b1I_ref[...], 0.0)
    w = wait(13); h2e = jnp.maximum(_dott(h1e, w) + b2I_ref[...], 0.0)
    w = wait(14); embI = _dott(h2e, w) + b3I_ref[...]
    w = wait(15); state = _dott(embI, w) + bsi_ref[...]

    whh_v = wait(16)
    glc = glc_ref[...]
    bhh = bhh_ref[...]

    def step(t, h):
        x_t = xs[pl.ds(t * B, B), :]
        hn = jnp.tanh(x_t + _dott(h, whh_v) + bhh)
        return jnp.where(glc > t, hn, h)

    h = jax.lax.fori_loop(0, L, step, state)

    woh = wait(17)
    out_ref[...] = _dott(h, woh) + outpart + bo_ref[...]


@functools.partial(jax.jit, static_argnames=())
def kernel(A_data, V_data, output_data, input_data, gnn_layers, layer_parameters,
           output_lengths, input_lengths, params):
    p = params
    f32 = jnp.float32
    gl = gnn_layers.astype(jnp.int32)
    lp = layer_parameters.astype(jnp.int32)

    def twtb(n):
        return jnp.stack([p[n + '_tw'], p[n + '_tb']], axis=0)

    def row(v):
        return v.reshape(1, -1)

    vspec = pl.BlockSpec(memory_space=pltpu.MemorySpace.VMEM)
    hspec = pl.BlockSpec(memory_space=pltpu.MemorySpace.HBM)

    small_ops = (
        A_data.transpose(1, 0, 2), V_data.transpose(1, 0, 2),
        output_data, input_data,
        gl.reshape(B, 1), gl.reshape(1, B), lp.reshape(1, B),
        output_lengths.astype(jnp.int32).reshape(B, 1),
        input_lengths.astype(jnp.int32).reshape(B, 1),
        twtb('A'), twtb('V'), twtb('out'), twtb('inp'),
        row(p['A_b1']), row(p['A_b2']), row(p['A_b3']),
        row(p['V_b1']), row(p['V_b2']), row(p['V_b3']),
        row(p['out_b1']), row(p['out_b2']), row(p['out_b3']),
        row(p['inp_b1']), row(p['inp_b2']), row(p['inp_b3']),
        row(p['bih']), row(p['bsi']), row(p['bhh']), row(p['bo']),
    )
    big_ops = (
        p['A_W1'], p['V_W1'], p['out_W1'], p['inp_W1'],
        p['A_W2'], p['A_W3'], p['V_W2'], p['V_W3'],
        p['out_W2'], p['out_W3'], p['inp_W2'], p['inp_W3'],
        p['Wih'], p['Wo'], p['Wsi'], p['Whh'],
    )

    return pl.pallas_call(
        _mega_kernel,
        out_shape=jax.ShapeDtypeStruct((B, OUT), f32),
        in_specs=[vspec] * len(small_ops) + [hspec] * len(big_ops),
        out_specs=vspec,
        scratch_shapes=[
            pltpu.VMEM((NBUF, H, H), f32),
            pltpu.VMEM((L * B, R), f32),
            pltpu.SemaphoreType.DMA((NBUF,)),
        ],
        compiler_params=pltpu.CompilerParams(
            vmem_limit_bytes=100 * 1024 * 1024,
        ),
    )(*small_ops, *big_ops)


# speedup vs baseline: 3.8423x; 1.0665x over previous
import functools

import jax
import jax.numpy as jnp
from jax.experimental import pallas as pl
from jax.experimental.pallas import tpu as pltpu

B, L, P, T = 16, 32, 32, 512
H, R, OUT = 1024, 1024, 1024
NBUF = 6

_DN = (((1,), (1,)), ((), ()))


def _dott(x, w):
    return jax.lax.dot_general(x, w, _DN, preferred_element_type=jnp.float32)


def _iota(shape, dim):
    return jax.lax.broadcasted_iota(jnp.int32, shape, dim)


def _mega_kernel(AT_ref, VT_ref, od_ref, id_ref,
                 glc_ref, glr_ref, lpr_ref, olen_ref, ilen_ref,
                 twtbA_ref, twtbV_ref, twtbO_ref, twtbI_ref,
                 b1A_ref, b2A_ref, b3A_ref, b1V_ref, b2V_ref, b3V_ref,
                 b1O_ref, b2O_ref, b3O_ref, b1I_ref, b2I_ref, b3I_ref,
                 bih_ref, bsi_ref, bhh_ref, bo_ref,
                 w1a, w1v, w1o, w1i, w2a, w3a, w2v, w3v,
                 w2o, w3o, w2i, w3i, wih, wo, wsi, whh,
                 out_ref, wbuf, wih_buf, wo_buf, xs, sems, sem_ih, sem_o):
    srcs = [w1a, w1v, w1o, w1i, w2a, w3a,
            w2v, w3v, w2o, w3o, w2i, w3i, wsi, whh]
    n_uses = len(srcs)

    def start(k):
        if k < n_uses:
            pltpu.make_async_copy(srcs[k], wbuf.at[k % NBUF], sems.at[k % NBUF]).start()

    def wait(k):
        pltpu.make_async_copy(srcs[k], wbuf.at[k % NBUF], sems.at[k % NBUF]).wait()
        return wbuf[k % NBUF]

    cp_ih = pltpu.make_async_copy(wih, wih_buf, sem_ih)
    cp_o = pltpu.make_async_copy(wo, wo_buf, sem_o)

    for k in range(NBUF):
        start(k)
    cp_ih.start()

    glr = glr_ref[...]
    lpr = lpr_ref[...]
    lmask = (_iota((L, B), 0) < glr).astype(jnp.float32)
    c_lb = lmask * lpr.astype(jnp.float32)
    pmask = (_iota((L, B, P), 2) < lpr[:, :, None]).astype(jnp.float32)
    sA = jnp.sum(AT_ref[...] * pmask, axis=2) * lmask
    sV = jnp.sum(VT_ref[...] * pmask, axis=2) * lmask

    olen = olen_ref[...]
    ilen = ilen_ref[...]
    omask = (_iota((B, T), 1) < olen).astype(jnp.float32)
    imask = (_iota((B, T), 1) < ilen).astype(jnp.float32)
    sO = jnp.sum(od_ref[...] * omask, axis=1, keepdims=True)
    sI = jnp.sum(id_ref[...] * imask, axis=1, keepdims=True)
    cO = olen.astype(jnp.float32)
    cI = ilen.astype(jnp.float32)

    w = wait(0); uvA = _dott(twtbA_ref[...], w); start(0 + NBUF)
    w = wait(1); uvV = _dott(twtbV_ref[...], w); start(1 + NBUF)
    w = wait(2); uvO = _dott(twtbO_ref[...], w); start(2 + NBUF)
    w = wait(3); uvI = _dott(twtbI_ref[...], w); start(3 + NBUF)
    cp_o.start()

    def h1_layer(s_lb, uv, b1):
        h = (s_lb[:, :, None] * uv[0:1, :][None]
             + c_lb[:, :, None] * uv[1:2, :][None] + b1[...][None])
        return jnp.maximum(h, 0.0).reshape(L * B, H)

    h1 = h1_layer(sA, uvA, b1A_ref)
    w = wait(4); h2 = jnp.maximum(_dott(h1, w) + b2A_ref[...], 0.0); start(4 + NBUF)
    w = wait(5); enc = _dott(h2, w) + b3A_ref[...]; start(5 + NBUF)
    cp_ih.wait()
    xs[...] = _dott(enc, wih_buf[:, :H]) + bih_ref[...]

    h1 = h1_layer(sV, uvV, b1V_ref)
    w = wait(6); h2 = jnp.maximum(_dott(h1, w) + b2V_ref[...], 0.0); start(6 + NBUF)
    w = wait(7); enc = _dott(h2, w) + b3V_ref[...]; start(7 + NBUF)
    xs[...] = xs[...] + _dott(enc, wih_buf[:, H:])

    h1e = jnp.maximum(sO * uvO[0:1, :] + cO * uvO[1:2, :] + b1O_ref[...], 0.0)
    w = wait(8); h2e = jnp.maximum(_dott(h1e, w) + b2O_ref[...], 0.0)
    w = wait(9); embO = _dott(h2e, w) + b3O_ref[...]
    cp_o.wait()
    outpart = _dott(embO, wo_buf[:, R:])

    h1e = jnp.maximum(sI * uvI[0:1, :] + cI * uvI[1:2, :] + b1I_ref[...], 0.0)
    w = wait(10); h2e = jnp.maximum(_dott(h1e, w) + b2I_ref[...], 0.0)
    w = wait(11); embI = _dott(h2e, w) + b3I_ref[...]
    w = wait(12); state = _dott(embI, w) + bsi_ref[...]

    whh_v = wait(13)
    glc = glc_ref[...]
    bhh = bhh_ref[...]
    h = state
    for t in range(L):
        x_t = xs[t * B:(t + 1) * B, :]
        hn = jnp.tanh(x_t + _dott(h, whh_v) + bhh)
        h = jnp.where(glc > t, hn, h)

    out_ref[...] = _dott(h, wo_buf[:, :R]) + outpart + bo_ref[...]


@functools.partial(jax.jit, static_argnames=())
def kernel(A_data, V_data, output_data, input_data, gnn_layers, layer_parameters,
           output_lengths, input_lengths, params):
    p = params
    f32 = jnp.float32
    gl = gnn_layers.astype(jnp.int32)
    lp = layer_parameters.astype(jnp.int32)

    def twtb(n):
        return jnp.stack([p[n + '_tw'], p[n + '_tb']], axis=0)

    def row(v):
        return v.reshape(1, -1)

    vspec = pl.BlockSpec(memory_space=pltpu.MemorySpace.VMEM)
    hspec = pl.BlockSpec(memory_space=pltpu.MemorySpace.HBM)

    small_ops = (
        A_data.transpose(1, 0, 2), V_data.transpose(1, 0, 2),
        output_data, input_data,
        gl.reshape(B, 1), gl.reshape(1, B), lp.reshape(1, B),
        output_lengths.astype(jnp.int32).reshape(B, 1),
        input_lengths.astype(jnp.int32).reshape(B, 1),
        twtb('A'), twtb('V'), twtb('out'), twtb('inp'),
        row(p['A_b1']), row(p['A_b2']), row(p['A_b3']),
        row(p['V_b1']), row(p['V_b2']), row(p['V_b3']),
        row(p['out_b1']), row(p['out_b2']), row(p['out_b3']),
        row(p['inp_b1']), row(p['inp_b2']), row(p['inp_b3']),
        row(p['bih']), row(p['bsi']), row(p['bhh']), row(p['bo']),
    )
    big_ops = (
        p['A_W1'], p['V_W1'], p['out_W1'], p['inp_W1'],
        p['A_W2'], p['A_W3'], p['V_W2'], p['V_W3'],
        p['out_W2'], p['out_W3'], p['inp_W2'], p['inp_W3'],
        p['Wih'], p['Wo'], p['Wsi'], p['Whh'],
    )

    return pl.pallas_call(
        _mega_kernel,
        out_shape=jax.ShapeDtypeStruct((B, OUT), f32),
        in_specs=[vspec] * len(small_ops) + [hspec] * len(big_ops),
        out_specs=vspec,
        scratch_shapes=[
            pltpu.VMEM((NBUF, H, H), f32),
            pltpu.VMEM((R, 2 * H), f32),
            pltpu.VMEM((OUT, 2 * H), f32),
            pltpu.VMEM((L * B, R), f32),
            pltpu.SemaphoreType.DMA((NBUF,)),
            pltpu.SemaphoreType.DMA,
            pltpu.SemaphoreType.DMA,
        ],
        compiler_params=pltpu.CompilerParams(
            vmem_limit_bytes=100 * 1024 * 1024,
        ),
    )(*small_ops, *big_ops)


# speedup vs baseline: 4.0703x; 1.0594x over previous
import functools

import jax
import jax.numpy as jnp
from jax.experimental import pallas as pl
from jax.experimental.pallas import tpu as pltpu

B, L, P, T = 16, 32, 32, 512
H, R, OUT = 1024, 1024, 1024
NBUF = 4

_DN = (((1,), (1,)), ((), ()))


def _dott(x, w):
    return jax.lax.dot_general(x, w, _DN, preferred_element_type=jnp.float32)


def _iota(shape, dim):
    return jax.lax.broadcasted_iota(jnp.int32, shape, dim)


def _mega_kernel(AT_ref, VT_ref, od_ref, id_ref,
                 glc_ref, glr_ref, lpr_ref, olen_ref, ilen_ref,
                 twtbA_ref, twtbV_ref, twtbO_ref, twtbI_ref,
                 b1A_ref, b2A_ref, b3A_ref, b1V_ref, b2V_ref, b3V_ref,
                 b1O_ref, b2O_ref, b3O_ref, b1I_ref, b2I_ref, b3I_ref,
                 bih_ref, bsi_ref, bhh_ref, bo_ref,
                 w1a, w1v, w1o, w1i, w2a, w3a, w2v, w3v,
                 w2o, w3o, w2i, w3i, wih, wo, wsi, whh,
                 out_ref, wbuf, wih_buf, wo_buf, xs, sems, sem_ih, sem_o):
    srcs = [w1a, w1v, w1i, w2a, w3a,
            w2v, w3v,
            w2i, w3i, wsi,
            whh,
            w1o, w2o, w3o]
    n_uses = len(srcs)

    cp_ih = pltpu.make_async_copy(wih, wih_buf, sem_ih)
    cp_o = pltpu.make_async_copy(wo, wo_buf, sem_o)

    def start(k):
        if k < n_uses:
            pltpu.make_async_copy(srcs[k], wbuf.at[k % NBUF], sems.at[k % NBUF]).start()

    def wait(k):
        pltpu.make_async_copy(srcs[k], wbuf.at[k % NBUF], sems.at[k % NBUF]).wait()
        return wbuf[k % NBUF]

    for k in range(NBUF):
        start(k)

    glr = glr_ref[...]
    lpr = lpr_ref[...]
    lmask = (_iota((L, B), 0) < glr).astype(jnp.float32)
    c_lb = lmask * lpr.astype(jnp.float32)
    pmask = (_iota((L, B, P), 2) < lpr[:, :, None]).astype(jnp.float32)
    sA = jnp.sum(AT_ref[...] * pmask, axis=2) * lmask
    sV = jnp.sum(VT_ref[...] * pmask, axis=2) * lmask

    olen = olen_ref[...]
    ilen = ilen_ref[...]
    omask = (_iota((B, T), 1) < olen).astype(jnp.float32)
    imask = (_iota((B, T), 1) < ilen).astype(jnp.float32)
    sO = jnp.sum(od_ref[...] * omask, axis=1, keepdims=True)
    sI = jnp.sum(id_ref[...] * imask, axis=1, keepdims=True)
    cO = olen.astype(jnp.float32)
    cI = ilen.astype(jnp.float32)

    def h1_layer(s_lb, uv, b1):
        h = (s_lb[:, :, None] * uv[0:1, :][None]
             + c_lb[:, :, None] * uv[1:2, :][None] + b1[...][None])
        return jnp.maximum(h, 0.0).reshape(L * B, H)

    w = wait(0); uvA = _dott(twtbA_ref[...], w); start(4)
    w = wait(1); uvV = _dott(twtbV_ref[...], w); cp_ih.start()
    w = wait(2); uvI = _dott(twtbI_ref[...], w); start(5)

    h1 = h1_layer(sA, uvA, b1A_ref)
    w = wait(3); h2 = jnp.maximum(_dott(h1, w) + b2A_ref[...], 0.0); start(6)
    w = wait(4); enc = _dott(h2, w) + b3A_ref[...]; start(7)
    cp_ih.wait()
    xs[...] = _dott(enc, wih_buf[:, :H]) + bih_ref[...]

    h1 = h1_layer(sV, uvV, b1V_ref)
    w = wait(5); h2 = jnp.maximum(_dott(h1, w) + b2V_ref[...], 0.0); start(8)
    w = wait(6); enc = _dott(h2, w) + b3V_ref[...]; start(9)
    xs[...] = xs[...] + _dott(enc, wih_buf[:, H:])

    h1e = jnp.maximum(sI * uvI[0:1, :] + cI * uvI[1:2, :] + b1I_ref[...], 0.0)
    w = wait(7); h2e = jnp.maximum(_dott(h1e, w) + b2I_ref[...], 0.0); start(10)
    w = wait(8); embI = _dott(h2e, w) + b3I_ref[...]; start(11)
    w = wait(9); state = _dott(embI, w) + bsi_ref[...]; start(12)

    whh_v = wait(10); start(13)
    cp_o.start()
    glc = glc_ref[...]
    bhh = bhh_ref[...]
    h = state
    for t in range(L):
        x_t = xs[t * B:(t + 1) * B, :]
        hn = jnp.tanh(x_t + _dott(h, whh_v) + bhh)
        h = jnp.where(glc > t, hn, h)

    w = wait(11); uvO = _dott(twtbO_ref[...], w)
    h1e = jnp.maximum(sO * uvO[0:1, :] + cO * uvO[1:2, :] + b1O_ref[...], 0.0)
    w = wait(12); h2e = jnp.maximum(_dott(h1e, w) + b2O_ref[...], 0.0)
    w = wait(13); embO = _dott(h2e, w) + b3O_ref[...]
    cp_o.wait()
    outpart = _dott(embO, wo_buf[:, R:])

    out_ref[...] = _dott(h, wo_buf[:, :R]) + outpart + bo_ref[...]


@functools.partial(jax.jit, static_argnames=())
def kernel(A_data, V_data, output_data, input_data, gnn_layers, layer_parameters,
           output_lengths, input_lengths, params):
    p = params
    f32 = jnp.float32
    gl = gnn_layers.astype(jnp.int32)
    lp = layer_parameters.astype(jnp.int32)

    def twtb(n):
        return jnp.stack([p[n + '_tw'], p[n + '_tb']], axis=0)

    def row(v):
        return v.reshape(1, -1)

    vspec = pl.BlockSpec(memory_space=pltpu.MemorySpace.VMEM)
    hspec = pl.BlockSpec(memory_space=pltpu.MemorySpace.HBM)

    small_ops = (
        A_data.transpose(1, 0, 2), V_data.transpose(1, 0, 2),
        output_data, input_data,
        gl.reshape(B, 1), gl.reshape(1, B), lp.reshape(1, B),
        output_lengths.astype(jnp.int32).reshape(B, 1),
        input_lengths.astype(jnp.int32).reshape(B, 1),
        twtb('A'), twtb('V'), twtb('out'), twtb('inp'),
        row(p['A_b1']), row(p['A_b2']), row(p['A_b3']),
        row(p['V_b1']), row(p['V_b2']), row(p['V_b3']),
        row(p['out_b1']), row(p['out_b2']), row(p['out_b3']),
        row(p['inp_b1']), row(p['inp_b2']), row(p['inp_b3']),
        row(p['bih']), row(p['bsi']), row(p['bhh']), row(p['bo']),
    )
    big_ops = (
        p['A_W1'], p['V_W1'], p['out_W1'], p['inp_W1'],
        p['A_W2'], p['A_W3'], p['V_W2'], p['V_W3'],
        p['out_W2'], p['out_W3'], p['inp_W2'], p['inp_W3'],
        p['Wih'], p['Wo'], p['Wsi'], p['Whh'],
    )

    return pl.pallas_call(
        _mega_kernel,
        out_shape=jax.ShapeDtypeStruct((B, OUT), f32),
        in_specs=[vspec] * len(small_ops) + [hspec] * len(big_ops),
        out_specs=vspec,
        scratch_shapes=[
            pltpu.VMEM((NBUF, H, H), f32),
            pltpu.VMEM((R, 2 * H), f32),
            pltpu.VMEM((OUT, 2 * H), f32),
            pltpu.VMEM((L * B, R), f32),
            pltpu.SemaphoreType.DMA((NBUF,)),
            pltpu.SemaphoreType.DMA,
            pltpu.SemaphoreType.DMA,
        ],
        compiler_params=pltpu.CompilerParams(
            vmem_limit_bytes=100 * 1024 * 1024,
        ),
    )(*small_ops, *big_ops)


# speedup vs baseline: 4.9261x; 1.2102x over previous
import functools

import jax
import jax.numpy as jnp
from jax.experimental import pallas as pl
from jax.experimental.pallas import tpu as pltpu

B, L, P, T = 16, 32, 32, 512
H, R, OUT = 1024, 1024, 1024
NBUF = 4

_DN = (((1,), (1,)), ((), ()))


def _dott(x, w):
    return jax.lax.dot_general(x, w, _DN, preferred_element_type=jnp.float32)


def _iota(shape, dim):
    return jax.lax.broadcasted_iota(jnp.int32, shape, dim)


def _row(ref):
    return ref[...].reshape(1, -1)


def _mega_kernel(A_ref, V_ref, od_ref, id_ref,
                 gl_ref, lp_ref, olen_ref, ilen_ref,
                 twA_ref, tbA_ref, twV_ref, tbV_ref,
                 twO_ref, tbO_ref, twI_ref, tbI_ref,
                 b1A_ref, b2A_ref, b3A_ref, b1V_ref, b2V_ref, b3V_ref,
                 b1O_ref, b2O_ref, b3O_ref, b1I_ref, b2I_ref, b3I_ref,
                 bih_ref, bsi_ref, bhh_ref, bo_ref,
                 w1a, w1v, w1o, w1i, w2a, w3a, w2v, w3v,
                 w2o, w3o, w2i, w3i, wih, wo, wsi, whh,
                 out_ref, wbuf, wih_buf, wo_buf, xs, sems, sem_ih, sem_o):
    srcs = [w1a, w1v, w1i, w2a, w3a,
            w2v, w3v,
            w2i, w3i, wsi,
            whh,
            w1o, w2o, w3o]
    n_uses = len(srcs)

    cp_ih = pltpu.make_async_copy(wih, wih_buf, sem_ih)
    cp_o = pltpu.make_async_copy(wo, wo_buf, sem_o)

    def start(k):
        if k < n_uses:
            pltpu.make_async_copy(srcs[k], wbuf.at[k % NBUF], sems.at[k % NBUF]).start()

    def wait(k):
        pltpu.make_async_copy(srcs[k], wbuf.at[k % NBUF], sems.at[k % NBUF]).wait()
        return wbuf[k % NBUF]

    for k in range(NBUF):
        start(k)

    glr = gl_ref[...].reshape(1, B)
    lpr = lp_ref[...].reshape(1, B)
    glc = gl_ref[...].reshape(B, 1)
    lmask = (_iota((L, B), 0) < glr).astype(jnp.float32)
    c_lb = lmask * lpr.astype(jnp.float32)
    pmaskT = (_iota((B, L, P), 2)
              < lp_ref[...].reshape(B, 1, 1)).astype(jnp.float32)
    sA = jnp.sum(A_ref[...] * pmaskT, axis=2).T * lmask
    sV = jnp.sum(V_ref[...] * pmaskT, axis=2).T * lmask

    olen = olen_ref[...].reshape(B, 1)
    ilen = ilen_ref[...].reshape(B, 1)
    omask = (_iota((B, T), 1) < olen).astype(jnp.float32)
    imask = (_iota((B, T), 1) < ilen).astype(jnp.float32)
    sO = jnp.sum(od_ref[...] * omask, axis=1, keepdims=True)
    sI = jnp.sum(id_ref[...] * imask, axis=1, keepdims=True)
    cO = olen.astype(jnp.float32)
    cI = ilen.astype(jnp.float32)

    def twtb(tw_ref, tb_ref):
        return jnp.concatenate([_row(tw_ref), _row(tb_ref)], axis=0)

    def h1_layer(s_lb, uv, b1_ref):
        h = (s_lb[:, :, None] * uv[0:1, :][None]
             + c_lb[:, :, None] * uv[1:2, :][None] + _row(b1_ref)[None])
        return jnp.maximum(h, 0.0).reshape(L * B, H)

    w = wait(0); uvA = _dott(twtb(twA_ref, tbA_ref), w); start(4)
    w = wait(1); uvV = _dott(twtb(twV_ref, tbV_ref), w); cp_ih.start()
    w = wait(2); uvI = _dott(twtb(twI_ref, tbI_ref), w); start(5)

    h1 = h1_layer(sA, uvA, b1A_ref)
    w = wait(3); h2 = jnp.maximum(_dott(h1, w) + _row(b2A_ref), 0.0); start(6)
    w = wait(4); enc = _dott(h2, w) + _row(b3A_ref); start(7)
    cp_ih.wait()
    xs[...] = _dott(enc, wih_buf[:, :H]) + _row(bih_ref)

    h1 = h1_layer(sV, uvV, b1V_ref)
    w = wait(5); h2 = jnp.maximum(_dott(h1, w) + _row(b2V_ref), 0.0); start(8)
    w = wait(6); enc = _dott(h2, w) + _row(b3V_ref); start(9)
    xs[...] = xs[...] + _dott(enc, wih_buf[:, H:])

    h1e = jnp.maximum(sI * uvI[0:1, :] + cI * uvI[1:2, :] + _row(b1I_ref), 0.0)
    w = wait(7); h2e = jnp.maximum(_dott(h1e, w) + _row(b2I_ref), 0.0); start(10)
    w = wait(8); embI = _dott(h2e, w) + _row(b3I_ref); start(11)
    w = wait(9); state = _dott(embI, w) + _row(bsi_ref); start(12)

    whh_v = wait(10); start(13)
    cp_o.start()
    bhh = _row(bhh_ref)
    h = state
    for t in range(L):
        x_t = xs[t * B:(t + 1) * B, :]
        hn = jnp.tanh(x_t + _dott(h, whh_v) + bhh)
        h = jnp.where(glc > t, hn, h)

    w = wait(11); uvO = _dott(twtb(twO_ref, tbO_ref), w)
    h1e = jnp.maximum(sO * uvO[0:1, :] + cO * uvO[1:2, :] + _row(b1O_ref), 0.0)
    w = wait(12); h2e = jnp.maximum(_dott(h1e, w) + _row(b2O_ref), 0.0)
    w = wait(13); embO = _dott(h2e, w) + _row(b3O_ref)
    cp_o.wait()
    outpart = _dott(embO, wo_buf[:, R:])

    out_ref[...] = _dott(h, wo_buf[:, :R]) + outpart + _row(bo_ref)


@functools.partial(jax.jit, static_argnames=())
def kernel(A_data, V_data, output_data, input_data, gnn_layers, layer_parameters,
           output_lengths, input_lengths, params):
    p = params
    f32 = jnp.float32

    vspec = pl.BlockSpec(memory_space=pltpu.MemorySpace.VMEM)
    hspec = pl.BlockSpec(memory_space=pltpu.MemorySpace.HBM)

    small_ops = (
        A_data, V_data, output_data, input_data,
        gnn_layers.astype(jnp.int32), layer_parameters.astype(jnp.int32),
        output_lengths.astype(jnp.int32), input_lengths.astype(jnp.int32),
        p['A_tw'], p['A_tb'], p['V_tw'], p['V_tb'],
        p['out_tw'], p['out_tb'], p['inp_tw'], p['inp_tb'],
        p['A_b1'], p['A_b2'], p['A_b3'],
        p['V_b1'], p['V_b2'], p['V_b3'],
        p['out_b1'], p['out_b2'], p['out_b3'],
        p['inp_b1'], p['inp_b2'], p['inp_b3'],
        p['bih'], p['bsi'], p['bhh'], p['bo'],
    )
    big_ops = (
        p['A_W1'], p['V_W1'], p['out_W1'], p['inp_W1'],
        p['A_W2'], p['A_W3'], p['V_W2'], p['V_W3'],
        p['out_W2'], p['out_W3'], p['inp_W2'], p['inp_W3'],
        p['Wih'], p['Wo'], p['Wsi'], p['Whh'],
    )

    return pl.pallas_call(
        _mega_kernel,
        out_shape=jax.ShapeDtypeStruct((B, OUT), f32),
        in_specs=[vspec] * len(small_ops) + [hspec] * len(big_ops),
        out_specs=vspec,
        scratch_shapes=[
            pltpu.VMEM((NBUF, H, H), f32),
            pltpu.VMEM((R, 2 * H), f32),
            pltpu.VMEM((OUT, 2 * H), f32),
            pltpu.VMEM((L * B, R), f32),
            pltpu.SemaphoreType.DMA((NBUF,)),
            pltpu.SemaphoreType.DMA,
            pltpu.SemaphoreType.DMA,
        ],
        compiler_params=pltpu.CompilerParams(
            vmem_limit_bytes=100 * 1024 * 1024,
        ),
    )(*small_ops, *big_ops)
